# baseline (speedup 1.0000x reference)
import jax
import jax.numpy as jnp
from jax import lax
from jax.experimental import pallas as pl
from jax.experimental.pallas import tpu as pltpu
from jax.experimental.pallas import tpu_sc as plsc

_NUM_ITEMS = 10000
_EMBED = 64
_BATCH = 16384
_NC = 2
_NS = 16
_NW = _NC * _NS
_BPW = _BATCH // _NW
_CHUNK = 128
_NCH = _BPW // _CHUNK
_GROUPS = _BPW // 16

_C1 = (1e-5 - 1.0) ** 2
_C2 = (1e-5) ** 2
_K1 = 0.1 * _C1
_K2 = 0.1 * _C2


def _sigmoid(x):
    return 1.0 / (1.0 + jnp.exp(-x))


def _gamma_body(users_r, pos_r, neg_r, gamma_r, gv_r,
                u_idx, p_idx, n_idx, u8, p1, n1, gbuf, gvals, semg):
    wid = lax.axis_index("s") * _NC + lax.axis_index("c")
    base = wid * _NCH
    pltpu.sync_copy(users_r.at[pl.ds(base, _NCH)], u_idx)
    pltpu.sync_copy(pos_r.at[pl.ds(base, _NCH)], p_idx)
    pltpu.sync_copy(neg_r.at[pl.ds(base, _NCH)], n_idx)

    lane = lax.iota(jnp.int32, 16)
    for r in range(_NCH):
        for c in range(_CHUNK // 16):
            sl = pl.ds(c * 16, 16)
            e0 = r * _CHUNK + c * 16
            plsc.store_scatter(u8, [(e0 + lane) * 8], u_idx[r, sl])
            p1[pl.ds(e0, 16)] = p_idx[r, sl]
            n1[pl.ds(e0, 16)] = n_idx[r, sl]

    def fetch_branch(items, out0):
        def issue(g, _):
            row0 = pl.multiple_of(g * 16, 16)
            c16 = lax.shift_right_logical(items[pl.ds(row0, 16)], 7) * _CHUNK
            for ln in range(16):
                c0 = pl.multiple_of(c16[ln], _CHUNK)
                e = row0 + ln
                col_view = gamma_r.at[:, pl.ds(c0, _CHUNK)]
                pltpu.async_copy(
                    col_view.at[u8.at[pl.ds(e * 8, 1)]],
                    gbuf.at[pl.ds(e, 1)], semg)
            return 0

        lax.fori_loop(0, _GROUPS, issue, 0)

        pltpu.make_async_copy(
            gamma_r.at[:, pl.ds(0, _CHUNK)].at[u8.at[pl.ds(0, _BPW)]],
            gbuf, semg).wait()

        for g in range(_GROUPS):
            row0 = g * 16
            rid = row0 + lane
            gvals[pl.ds(out0 + row0, 16)] = plsc.load_gather(
                gbuf, [rid, items[pl.ds(row0, 16)] & 127])

    fetch_branch(p1, 0)
    fetch_branch(n1, _BPW)

    pltpu.sync_copy(gvals, gv_r.at[pl.ds(wid * 2 * _BPW, 2 * _BPW)])


_gamma_call = pl.kernel(
    _gamma_body,
    out_type=jax.ShapeDtypeStruct((_NW * 2 * _BPW,), jnp.float32),
    mesh=plsc.VectorSubcoreMesh(core_axis_name="c", subcore_axis_name="s"),
    compiler_params=pltpu.CompilerParams(
        needs_layout_passes=False, use_tc_tiling_on_sc=True),
    scratch_types=[
        pltpu.VMEM((_NCH, _CHUNK), jnp.int32),
        pltpu.VMEM((_NCH, _CHUNK), jnp.int32),
        pltpu.VMEM((_NCH, _CHUNK), jnp.int32),
        pltpu.VMEM((8 * _BPW,), jnp.int32),
        pltpu.VMEM((_BPW,), jnp.int32),
        pltpu.VMEM((_BPW,), jnp.int32),
        pltpu.VMEM((_BPW, _CHUNK), jnp.float32),
        pltpu.VMEM((2 * _BPW,), jnp.float32),
        pltpu.SemaphoreType.DMA,
    ],
)


def _loss_body(users_r, pos_r, neg_r, ue_r, ie_r, gv_r, out_r,
               u_idx, p_idx, n_idx, urows, prows, nrows, gv1,
               pstage, nstage, lout, sem):
    wid = lax.axis_index("s") * _NC + lax.axis_index("c")
    base = wid * _NCH
    pltpu.sync_copy(users_r.at[pl.ds(base, _NCH)], u_idx)
    pltpu.sync_copy(pos_r.at[pl.ds(base, _NCH)], p_idx)
    pltpu.sync_copy(neg_r.at[pl.ds(base, _NCH)], n_idx)

    copies = []
    for j in range(_NCH):
        dst = pl.ds(j * _CHUNK, _CHUNK)
        copies.append(pltpu.async_copy(ue_r.at[u_idx.at[j]], urows.at[dst], sem[j]))
        copies.append(pltpu.async_copy(ie_r.at[p_idx.at[j]], prows.at[dst], sem[j]))
        copies.append(pltpu.async_copy(ie_r.at[n_idx.at[j]], nrows.at[dst], sem[j]))

    pltpu.sync_copy(gv_r.at[pl.ds(wid * 2 * _BPW, 2 * _BPW)], gv1)

    lane = lax.iota(jnp.int32, 16)
    zero = jnp.zeros((16,), jnp.float32)

    last = jnp.full((16,), 15, jnp.int32)

    def loss_group(g, acc):
        row0 = pl.multiple_of(g * 16, 16)
        for ln in range(16):
            e = row0 + ln
            pa = zero
            na = zero
            for c in range(_EMBED // 16):
                sl = pl.ds(c * 16, 16)
                uv = urows[e, sl]
                pa = pa + uv * prows[e, sl]
                na = na + uv * nrows[e, sl]
            pstage[ln, :] = plsc.cumsum(pa)
            nstage[ln, :] = plsc.cumsum(na)
        pa = plsc.load_gather(pstage, [lane, last])
        na = plsc.load_gather(nstage, [lane, last])
        ps = _sigmoid(pa)
        ns = _sigmoid(na)
        pg = _sigmoid(gv1[pl.ds(row0, 16)])
        ng = _sigmoid(gv1[pl.ds(_BPW + row0, 16)])
        t = ps - 1.0
        return acc + (pg * (t * t) + ng * (ns * ns)
                      + _K1 * (1.0 - pg) + _K2 * (1.0 - ng))

    acc = zero
    gpc = _GROUPS // _NCH
    for j in range(_NCH):
        for cp in copies[3 * j:3 * j + 3]:
            cp.wait()
        acc = lax.fori_loop(j * gpc, (j + 1) * gpc, loss_group, acc)
    lout[...] = acc
    pltpu.sync_copy(lout, out_r.at[wid])


_loss_call = pl.kernel(
    _loss_body,
    out_type=jax.ShapeDtypeStruct((_NW, 16), jnp.float32),
    mesh=plsc.VectorSubcoreMesh(core_axis_name="c", subcore_axis_name="s"),
    compiler_params=pltpu.CompilerParams(
        needs_layout_passes=False, use_tc_tiling_on_sc=False),
    scratch_types=[
        pltpu.VMEM((_NCH, _CHUNK), jnp.int32),
        pltpu.VMEM((_NCH, _CHUNK), jnp.int32),
        pltpu.VMEM((_NCH, _CHUNK), jnp.int32),
        pltpu.VMEM((_BPW, _EMBED), jnp.float32),
        pltpu.VMEM((_BPW, _EMBED), jnp.float32),
        pltpu.VMEM((_BPW, _EMBED), jnp.float32),
        pltpu.VMEM((2 * _BPW,), jnp.float32),
        pltpu.VMEM((16, 16), jnp.float32),
        pltpu.VMEM((16, 16), jnp.float32),
        pltpu.VMEM((16,), jnp.float32),
        [pltpu.SemaphoreType.DMA] * _NCH,
    ],
)


def kernel(users, positive_items, negative_items, user_embedding,
           item_embedding, gamma):
    u2 = users.astype(jnp.int32).reshape(_NW * _NCH, _CHUNK)
    p2 = positive_items.astype(jnp.int32).reshape(_NW * _NCH, _CHUNK)
    n2 = negative_items.astype(jnp.int32).reshape(_NW * _NCH, _CHUNK)
    gvals = _gamma_call(u2, p2, n2, gamma)
    parts = _loss_call(u2, p2, n2, user_embedding, item_embedding, gvals)
    return jnp.sum(parts) / jnp.float32(2 * _BATCH)


# speedup vs baseline: 1.0728x; 1.0728x over previous
import jax
import jax.numpy as jnp
from jax import lax
from jax.experimental import pallas as pl
from jax.experimental.pallas import tpu as pltpu
from jax.experimental.pallas import tpu_sc as plsc

_NUM_ITEMS = 10000
_EMBED = 64
_BATCH = 16384
_NC = 2
_NS = 16
_NW = _NC * _NS
_BPW = _BATCH // _NW
_CHUNK = 128
_NCH = _BPW // _CHUNK
_GROUPS = _BPW // 16

_C1 = (1e-5 - 1.0) ** 2
_C2 = (1e-5) ** 2
_K1 = 0.1 * _C1
_K2 = 0.1 * _C2


def _sigmoid(x):
    return 1.0 / (1.0 + jnp.exp(-x))


def _gamma_body(users_r, pos_r, neg_r, gamma_r, gv_r,
                u_idx, p_idx, n_idx, u8, p1, n1, gbuf, gvals, semg):
    wid = lax.axis_index("s") * _NC + lax.axis_index("c")
    base = wid * _NCH
    pltpu.sync_copy(users_r.at[pl.ds(base, _NCH)], u_idx)
    pltpu.sync_copy(pos_r.at[pl.ds(base, _NCH)], p_idx)
    pltpu.sync_copy(neg_r.at[pl.ds(base, _NCH)], n_idx)

    lane = lax.iota(jnp.int32, 16)
    for r in range(_NCH):
        for c in range(_CHUNK // 16):
            sl = pl.ds(c * 16, 16)
            e0 = r * _CHUNK + c * 16
            plsc.store_scatter(u8, [(e0 + lane) * 8], u_idx[r, sl])
            p1[pl.ds(e0, 16)] = p_idx[r, sl]
            n1[pl.ds(e0, 16)] = n_idx[r, sl]

    def fetch_branch(items, out0):
        def issue(g, _):
            row0 = pl.multiple_of(g * 16, 16)
            c16 = lax.shift_right_logical(items[pl.ds(row0, 16)], 7) * _CHUNK
            for ln in range(16):
                c0 = pl.multiple_of(c16[ln], _CHUNK)
                e = row0 + ln
                col_view = gamma_r.at[:, pl.ds(c0, _CHUNK)]
                pltpu.async_copy(
                    col_view.at[u8.at[pl.ds(e * 8, 1)]],
                    gbuf.at[pl.ds(e, 1)], semg)
            return 0

        lax.fori_loop(0, _GROUPS, issue, 0)

        pltpu.make_async_copy(
            gamma_r.at[:, pl.ds(0, _CHUNK)].at[u8.at[pl.ds(0, _BPW)]],
            gbuf, semg).wait()

        for g in range(_GROUPS):
            row0 = g * 16
            rid = row0 + lane
            gvals[pl.ds(out0 + row0, 16)] = plsc.load_gather(
                gbuf, [rid, items[pl.ds(row0, 16)] & 127])

    fetch_branch(p1, 0)
    fetch_branch(n1, _BPW)

    pltpu.sync_copy(gvals, gv_r.at[pl.ds(wid * 2 * _BPW, 2 * _BPW)])


_gamma_call = pl.kernel(
    _gamma_body,
    out_type=jax.ShapeDtypeStruct((_NW * 2 * _BPW,), jnp.float32),
    mesh=plsc.VectorSubcoreMesh(core_axis_name="c", subcore_axis_name="s"),
    compiler_params=pltpu.CompilerParams(
        needs_layout_passes=False, use_tc_tiling_on_sc=True),
    scratch_types=[
        pltpu.VMEM((_NCH, _CHUNK), jnp.int32),
        pltpu.VMEM((_NCH, _CHUNK), jnp.int32),
        pltpu.VMEM((_NCH, _CHUNK), jnp.int32),
        pltpu.VMEM((8 * _BPW,), jnp.int32),
        pltpu.VMEM((_BPW,), jnp.int32),
        pltpu.VMEM((_BPW,), jnp.int32),
        pltpu.VMEM((_BPW, _CHUNK), jnp.float32),
        pltpu.VMEM((2 * _BPW,), jnp.float32),
        pltpu.SemaphoreType.DMA,
    ],
)


def _loss_body(users_r, pos_r, neg_r, ue_r, ie_r, gv_r, out_r,
               u_idx, p_idx, n_idx, urows, prows, nrows, gv1,
               pstage, nstage, lout, sem):
    wid = lax.axis_index("s") * _NC + lax.axis_index("c")
    base = wid * _NCH
    pltpu.sync_copy(users_r.at[pl.ds(base, _NCH)], u_idx)
    pltpu.sync_copy(pos_r.at[pl.ds(base, _NCH)], p_idx)
    pltpu.sync_copy(neg_r.at[pl.ds(base, _NCH)], n_idx)

    copies = []
    for j in range(_NCH):
        dst = pl.ds(j * _CHUNK, _CHUNK)
        copies.append(pltpu.async_copy(ue_r.at[u_idx.at[j]], urows.at[dst], sem))
        copies.append(pltpu.async_copy(ie_r.at[p_idx.at[j]], prows.at[dst], sem))
        copies.append(pltpu.async_copy(ie_r.at[n_idx.at[j]], nrows.at[dst], sem))

    pltpu.sync_copy(gv_r.at[pl.ds(wid * 2 * _BPW, 2 * _BPW)], gv1)
    for cp in copies:
        cp.wait()

    lane = lax.iota(jnp.int32, 16)
    zero = jnp.zeros((16,), jnp.float32)

    last = jnp.full((16,), 15, jnp.int32)

    def loss_group(g, acc):
        row0 = pl.multiple_of(g * 16, 16)
        for ln in range(16):
            e = row0 + ln
            pa = zero
            na = zero
            for c in range(_EMBED // 16):
                sl = pl.ds(c * 16, 16)
                uv = urows[e, sl]
                pa = pa + uv * prows[e, sl]
                na = na + uv * nrows[e, sl]
            pstage[ln, :] = plsc.cumsum(pa)
            nstage[ln, :] = plsc.cumsum(na)
        pa = plsc.load_gather(pstage, [lane, last])
        na = plsc.load_gather(nstage, [lane, last])
        ps = _sigmoid(pa)
        ns = _sigmoid(na)
        pg = _sigmoid(gv1[pl.ds(row0, 16)])
        ng = _sigmoid(gv1[pl.ds(_BPW + row0, 16)])
        t = ps - 1.0
        return acc + (pg * (t * t) + ng * (ns * ns)
                      + _K1 * (1.0 - pg) + _K2 * (1.0 - ng))

    acc = lax.fori_loop(0, _GROUPS, loss_group, zero)
    lout[...] = acc
    pltpu.sync_copy(lout, out_r.at[wid])


_loss_call = pl.kernel(
    _loss_body,
    out_type=jax.ShapeDtypeStruct((_NW, 16), jnp.float32),
    mesh=plsc.VectorSubcoreMesh(core_axis_name="c", subcore_axis_name="s"),
    compiler_params=pltpu.CompilerParams(
        needs_layout_passes=False, use_tc_tiling_on_sc=False),
    scratch_types=[
        pltpu.VMEM((_NCH, _CHUNK), jnp.int32),
        pltpu.VMEM((_NCH, _CHUNK), jnp.int32),
        pltpu.VMEM((_NCH, _CHUNK), jnp.int32),
        pltpu.VMEM((_BPW, _EMBED), jnp.float32),
        pltpu.VMEM((_BPW, _EMBED), jnp.float32),
        pltpu.VMEM((_BPW, _EMBED), jnp.float32),
        pltpu.VMEM((2 * _BPW,), jnp.float32),
        pltpu.VMEM((16, 16), jnp.float32),
        pltpu.VMEM((16, 16), jnp.float32),
        pltpu.VMEM((16,), jnp.float32),
        pltpu.SemaphoreType.DMA,
    ],
)


def kernel(users, positive_items, negative_items, user_embedding,
           item_embedding, gamma):
    u2 = users.astype(jnp.int32).reshape(_NW * _NCH, _CHUNK)
    p2 = positive_items.astype(jnp.int32).reshape(_NW * _NCH, _CHUNK)
    n2 = negative_items.astype(jnp.int32).reshape(_NW * _NCH, _CHUNK)
    gvals = _gamma_call(u2, p2, n2, gamma)
    parts = _loss_call(u2, p2, n2, user_embedding, item_embedding, gvals)
    return jnp.sum(parts) / jnp.float32(2 * _BATCH)


# speedup vs baseline: 1.0750x; 1.0021x over previous
import jax
import jax.numpy as jnp
from jax import lax
from jax.experimental import pallas as pl
from jax.experimental.pallas import tpu as pltpu
from jax.experimental.pallas import tpu_sc as plsc

_NUM_ITEMS = 10000
_EMBED = 64
_BATCH = 16384
_NC = 2
_NS = 16
_NW = _NC * _NS
_BPW = _BATCH // _NW
_CHUNK = 128
_NCH = _BPW // _CHUNK
_GROUPS = _BPW // 16

_C1 = (1e-5 - 1.0) ** 2
_C2 = (1e-5) ** 2
_K1 = 0.1 * _C1
_K2 = 0.1 * _C2


def _sigmoid(x):
    return 1.0 / (1.0 + jnp.exp(-x))


def _gamma_body(users_r, pos_r, neg_r, gamma_r, gv_r,
                u_idx, p_idx, n_idx, u8, p1, n1, gbuf, gvals, semg):
    wid = lax.axis_index("s") * _NC + lax.axis_index("c")
    base = wid * _NCH
    pltpu.sync_copy(users_r.at[pl.ds(base, _NCH)], u_idx)
    pltpu.sync_copy(pos_r.at[pl.ds(base, _NCH)], p_idx)
    pltpu.sync_copy(neg_r.at[pl.ds(base, _NCH)], n_idx)

    lane = lax.iota(jnp.int32, 16)
    for r in range(_NCH):
        for c in range(_CHUNK // 16):
            sl = pl.ds(c * 16, 16)
            e0 = r * _CHUNK + c * 16
            plsc.store_scatter(u8, [(e0 + lane) * 8], u_idx[r, sl])
            p1[pl.ds(e0, 16)] = p_idx[r, sl]
            n1[pl.ds(e0, 16)] = n_idx[r, sl]

    def fetch_branch(items, out0):
        def issue(g, _):
            row0 = pl.multiple_of(g * 16, 16)
            c16 = lax.shift_right_logical(items[pl.ds(row0, 16)], 7) * _CHUNK
            for ln in range(16):
                c0 = pl.multiple_of(c16[ln], _CHUNK)
                e = row0 + ln
                col_view = gamma_r.at[:, pl.ds(c0, _CHUNK)]
                pltpu.async_copy(
                    col_view.at[u8.at[pl.ds(e * 8, 1)]],
                    gbuf.at[pl.ds(e, 1)], semg)
            return 0

        lax.fori_loop(0, _GROUPS, issue, 0)

        pltpu.make_async_copy(
            gamma_r.at[:, pl.ds(0, _CHUNK)].at[u8.at[pl.ds(0, _BPW)]],
            gbuf, semg).wait()

        for g in range(_GROUPS):
            row0 = g * 16
            rid = row0 + lane
            gvals[pl.ds(out0 + row0, 16)] = plsc.load_gather(
                gbuf, [rid, items[pl.ds(row0, 16)] & 127])

    fetch_branch(p1, 0)
    fetch_branch(n1, _BPW)

    pltpu.sync_copy(gvals, gv_r.at[pl.ds(wid * 2 * _BPW, 2 * _BPW)])


_gamma_call = pl.kernel(
    _gamma_body,
    out_type=jax.ShapeDtypeStruct((_NW * 2 * _BPW,), jnp.float32),
    mesh=plsc.VectorSubcoreMesh(core_axis_name="c", subcore_axis_name="s"),
    compiler_params=pltpu.CompilerParams(
        needs_layout_passes=False, use_tc_tiling_on_sc=True),
    scratch_types=[
        pltpu.VMEM((_NCH, _CHUNK), jnp.int32),
        pltpu.VMEM((_NCH, _CHUNK), jnp.int32),
        pltpu.VMEM((_NCH, _CHUNK), jnp.int32),
        pltpu.VMEM((8 * _BPW,), jnp.int32),
        pltpu.VMEM((_BPW,), jnp.int32),
        pltpu.VMEM((_BPW,), jnp.int32),
        pltpu.VMEM((_BPW, _CHUNK), jnp.float32),
        pltpu.VMEM((2 * _BPW,), jnp.float32),
        pltpu.SemaphoreType.DMA,
    ],
)


def _loss_body(users_r, pos_r, neg_r, ue_r, ie_r, gv_r, out_r,
               u_idx, p_idx, n_idx, urows, prows, nrows, gv1,
               pstage, nstage, lout, sem):
    wid = lax.axis_index("s") * _NC + lax.axis_index("c")
    base = wid * _NCH
    pltpu.sync_copy(users_r.at[pl.ds(base, _NCH)], u_idx)
    pltpu.sync_copy(pos_r.at[pl.ds(base, _NCH)], p_idx)
    pltpu.sync_copy(neg_r.at[pl.ds(base, _NCH)], n_idx)

    copies = []
    for j in range(_NCH):
        dst = pl.ds(j * _CHUNK, _CHUNK)
        copies.append(pltpu.async_copy(ue_r.at[u_idx.at[j]], urows.at[dst], sem))
        copies.append(pltpu.async_copy(ie_r.at[p_idx.at[j]], prows.at[dst], sem))
        copies.append(pltpu.async_copy(ie_r.at[n_idx.at[j]], nrows.at[dst], sem))

    pltpu.sync_copy(gv_r.at[pl.ds(wid * 2 * _BPW, 2 * _BPW)], gv1)
    for cp in copies:
        cp.wait()

    lane = lax.iota(jnp.int32, 16)
    zero = jnp.zeros((16,), jnp.float32)

    last = jnp.full((16,), 15, jnp.int32)

    def loss_group(g, acc):
        row0 = pl.multiple_of(g * 16, 16)
        for ln in range(16):
            e = row0 + ln
            pa = zero
            na = zero
            for c in range(_EMBED // 16):
                sl = pl.ds(c * 16, 16)
                uv = urows[e, sl]
                pa = pa + uv * prows[e, sl]
                na = na + uv * nrows[e, sl]
            pstage[ln, pl.ds(0, 16)] = plsc.cumsum(pa)
            nstage[ln, pl.ds(0, 16)] = plsc.cumsum(na)
        pa = plsc.load_gather(pstage, [lane, last])
        na = plsc.load_gather(nstage, [lane, last])
        ps = _sigmoid(pa)
        ns = _sigmoid(na)
        pg = _sigmoid(gv1[pl.ds(row0, 16)])
        ng = _sigmoid(gv1[pl.ds(_BPW + row0, 16)])
        t = ps - 1.0
        return acc + (pg * (t * t) + ng * (ns * ns)
                      + _K1 * (1.0 - pg) + _K2 * (1.0 - ng))

    acc = lax.fori_loop(0, _GROUPS, loss_group, zero)
    lout[...] = acc
    pltpu.sync_copy(lout, out_r.at[wid])


_loss_call = pl.kernel(
    _loss_body,
    out_type=jax.ShapeDtypeStruct((_NW, 16), jnp.float32),
    mesh=plsc.VectorSubcoreMesh(core_axis_name="c", subcore_axis_name="s"),
    compiler_params=pltpu.CompilerParams(
        needs_layout_passes=False, use_tc_tiling_on_sc=False),
    scratch_types=[
        pltpu.VMEM((_NCH, _CHUNK), jnp.int32),
        pltpu.VMEM((_NCH, _CHUNK), jnp.int32),
        pltpu.VMEM((_NCH, _CHUNK), jnp.int32),
        pltpu.VMEM((_BPW, _EMBED), jnp.float32),
        pltpu.VMEM((_BPW, _EMBED), jnp.float32),
        pltpu.VMEM((_BPW, _EMBED), jnp.float32),
        pltpu.VMEM((2 * _BPW,), jnp.float32),
        pltpu.VMEM((16, 17), jnp.float32),
        pltpu.VMEM((16, 17), jnp.float32),
        pltpu.VMEM((16,), jnp.float32),
        pltpu.SemaphoreType.DMA,
    ],
)


def kernel(users, positive_items, negative_items, user_embedding,
           item_embedding, gamma):
    u2 = users.astype(jnp.int32).reshape(_NW * _NCH, _CHUNK)
    p2 = positive_items.astype(jnp.int32).reshape(_NW * _NCH, _CHUNK)
    n2 = negative_items.astype(jnp.int32).reshape(_NW * _NCH, _CHUNK)
    gvals = _gamma_call(u2, p2, n2, gamma)
    parts = _loss_call(u2, p2, n2, user_embedding, item_embedding, gvals)
    return jnp.sum(parts) / jnp.float32(2 * _BATCH)
